# baseline (speedup 1.0000x reference)
import functools

import jax
import jax.numpy as jnp
from jax import lax
from jax.experimental import pallas as pl
from jax.experimental.pallas import tpu as pltpu
from jax.experimental.pallas import tpu_sc as plsc

_NBUF = 4


def _pick_chunk(n):
    for c in (800, 200, 8):
        if n % c == 0:
            return c
    return n


def _argmax_kernel(a_hbm, g_hbm, o_hbm, a_buf, g_buf, ids, macc, iacc,
                   a_sem, g_sem, o_sem, *, n, m, chunk):
    nchunks = n // chunk

    def a_copy(c, s):
        return pltpu.make_async_copy(
            a_hbm.at[pl.ds(c * chunk, chunk), :], a_buf.at[s], a_sem.at[s])

    def g_copy(c, s):
        return pltpu.make_async_copy(
            g_hbm.at[pl.ds(c * chunk, chunk), :], g_buf.at[s], g_sem.at[s])

    ids[...] = jax.lax.broadcasted_iota(jnp.int32, (chunk, m), 0)
    macc[...] = jnp.full((1, m), -jnp.inf, jnp.float32)
    iacc[...] = jnp.zeros((1, m), jnp.int32)

    for s in range(min(_NBUF, nchunks)):
        a_copy(s, s).start()
        g_copy(s, s).start()

    def body(i, carry):
        s = jax.lax.rem(i, _NBUF)
        a_copy(i, s).wait()
        g_copy(i, s).wait()

        x = a_buf[s] + g_buf[s]
        bm = jnp.max(x, axis=0, keepdims=True)
        bi = jnp.min(jnp.where(x >= bm, ids[...], jnp.int32(n)), axis=0,
                     keepdims=True) + i * chunk
        better = bm > macc[...]
        iacc[...] = jnp.where(better, bi, iacc[...])
        macc[...] = jnp.maximum(bm, macc[...])

        @pl.when(i + _NBUF < nchunks)
        def _():
            a_copy(i + _NBUF, s).start()
            g_copy(i + _NBUF, s).start()

        return carry

    jax.lax.fori_loop(0, nchunks, body, 0)
    pltpu.make_async_copy(iacc, o_hbm, o_sem).start()
    pltpu.make_async_copy(iacc, o_hbm, o_sem).wait()


def _tc_argmax(at, gt):
    n, m = at.shape
    chunk = _pick_chunk(n)
    buf = lambda: pltpu.VMEM((_NBUF, chunk, m), jnp.float32)
    return pl.pallas_call(
        functools.partial(_argmax_kernel, n=n, m=m, chunk=chunk),
        in_specs=[
            pl.BlockSpec(memory_space=pl.ANY),
            pl.BlockSpec(memory_space=pl.ANY),
        ],
        out_specs=pl.BlockSpec(memory_space=pl.ANY),
        out_shape=jax.ShapeDtypeStruct((1, m), jnp.int32),
        scratch_shapes=[
            buf(), buf(),
            pltpu.VMEM((chunk, m), jnp.int32),
            pltpu.VMEM((1, m), jnp.float32),
            pltpu.VMEM((1, m), jnp.int32),
            pltpu.SemaphoreType.DMA((_NBUF,)),
            pltpu.SemaphoreType.DMA((_NBUF,)),
            pltpu.SemaphoreType.DMA,
        ],
    )(at, gt)


_ZROWS = 8


def _sc_fill(n, m):
    info = plsc.get_sparse_core_info()
    nworkers = info.num_cores * info.num_subcores
    nchunks = n // _ZROWS
    per_w = -(-nchunks // nworkers)
    mesh = plsc.VectorSubcoreMesh(core_axis_name="c", subcore_axis_name="s")

    @functools.partial(
        pl.kernel, mesh=mesh,
        out_type=jax.ShapeDtypeStruct((n, m), jnp.float32),
        scratch_types=[
            pltpu.VMEM((_ZROWS, m), jnp.float32),
            pltpu.SemaphoreType.DMA,
        ],
    )
    def zfill(out_hbm, zbuf, sem):
        wid = lax.axis_index("s") * info.num_cores + lax.axis_index("c")
        zero16 = jnp.zeros((16,), jnp.float32)
        for r in range(_ZROWS):
            for l in range(m // 16):
                zbuf[r, pl.ds(l * 16, 16)] = zero16

        def fire(t, carry):
            c = wid + t * nworkers

            @pl.when(c < nchunks)
            def _():
                pltpu.make_async_copy(
                    zbuf, out_hbm.at[pl.ds(c * _ZROWS, _ZROWS), :],
                    sem).start()
            return carry

        jax.lax.fori_loop(0, per_w, fire, 0)

        def drain(t, carry):
            c = wid + t * nworkers

            @pl.when(c < nchunks)
            def _():
                pltpu.make_async_copy(
                    zbuf, out_hbm.at[pl.ds(c * _ZROWS, _ZROWS), :],
                    sem).wait()
            return carry

        jax.lax.fori_loop(0, per_w, drain, 0)

    return zfill()


def _patch_kernel(idx_sm, filled_hbm, idxv_ref, o_hbm, tbuf, p_sem, *, m):
    del filled_hbm
    ring = tbuf.shape[0]
    siota = jax.lax.broadcasted_iota(jnp.int32, (8, 128), 0)

    def fire(j, carry):
        s = jax.lax.rem(j, ring)
        r = idx_sm[j]
        q8 = pl.multiple_of((r // 8) * 8, 8)
        t128 = pl.multiple_of((j // 128) * 128, 128)

        @pl.when(j >= ring)
        def _():
            pltpu.make_async_copy(
                tbuf.at[s], o_hbm.at[pl.ds(q8, 8), pl.ds(t128, 128)],
                p_sem).wait()

        tile = idxv_ref[:, pl.ds(t128, 128)] == (q8 + siota)
        tbuf[s] = tile.astype(jnp.float32)
        pltpu.make_async_copy(
            tbuf.at[s], o_hbm.at[pl.ds(q8, 8), pl.ds(t128, 128)],
            p_sem).start()
        return carry

    jax.lax.fori_loop(0, m, fire, 0)

    def drain(t, carry):
        j = m - tbuf.shape[0] + t
        s = jax.lax.rem(j, ring)
        r = idx_sm[j]
        q8 = pl.multiple_of((r // 8) * 8, 8)
        t128 = pl.multiple_of((j // 128) * 128, 128)
        pltpu.make_async_copy(
            tbuf.at[s], o_hbm.at[pl.ds(q8, 8), pl.ds(t128, 128)],
            p_sem).wait()
        return carry

    jax.lax.fori_loop(0, tbuf.shape[0], drain, 0)


def _tc_patch(filled, idx2d):
    n, m = filled.shape
    grid_spec = pltpu.PrefetchScalarGridSpec(
        num_scalar_prefetch=1,
        grid=(1,),
        in_specs=[
            pl.BlockSpec(memory_space=pl.ANY),
            pl.BlockSpec((1, m), lambda i, ref: (0, 0)),
        ],
        out_specs=pl.BlockSpec(memory_space=pl.ANY),
        scratch_shapes=[
            pltpu.VMEM((8, 8, 128), jnp.float32),
            pltpu.SemaphoreType.DMA,
        ],
    )
    return pl.pallas_call(
        functools.partial(_patch_kernel, m=m),
        grid_spec=grid_spec,
        out_shape=jax.ShapeDtypeStruct((n, m), jnp.float32),
        input_output_aliases={1: 0},
    )(idx2d.reshape(m), filled, idx2d)


def kernel(alpha, gumbel, tau):
    del tau
    at, gt = alpha.T, gumbel.T
    n, m = at.shape
    idx = _tc_argmax(at, gt)
    filled = _sc_fill(n, m)
    out_t = _tc_patch(filled, idx)
    return out_t.T


# speedup vs baseline: 1.3053x; 1.3053x over previous
import functools

import jax
import jax.numpy as jnp
from jax.experimental import pallas as pl
from jax.experimental.pallas import tpu as pltpu

_NBUF_IN = 4
_NBUF_OUT = 4


def _pick_chunk(n):
    for c in (800, 200, 8):
        if n % c == 0:
            return c
    return n


def _ring_kernel(a_hbm, g_hbm, o_hbm, a_buf, g_buf, o_buf, ids, macc, iacc,
                 a_sem, g_sem, o_sem, *, n, m, chunk):
    nchunks = n // chunk

    def a_copy(c, s):
        return pltpu.make_async_copy(
            a_hbm.at[pl.ds(c * chunk, chunk), :], a_buf.at[s], a_sem.at[s])

    def g_copy(c, s):
        return pltpu.make_async_copy(
            g_hbm.at[pl.ds(c * chunk, chunk), :], g_buf.at[s], g_sem.at[s])

    def o_copy(c, s):
        return pltpu.make_async_copy(
            o_buf.at[s], o_hbm.at[pl.ds(c * chunk, chunk), :], o_sem.at[s])

    ids[...] = jax.lax.broadcasted_iota(jnp.int32, (chunk, m), 0)
    macc[...] = jnp.full((1, m), -jnp.inf, jnp.float32)
    iacc[...] = jnp.zeros((1, m), jnp.int32)

    for s in range(min(_NBUF_IN, nchunks)):
        a_copy(s, s).start()
        g_copy(s, s).start()

    def body_a(i, carry):
        s = jax.lax.rem(i, _NBUF_IN)
        a_copy(i, s).wait()
        g_copy(i, s).wait()

        x = a_buf[s] + g_buf[s]
        bm = jnp.max(x, axis=0, keepdims=True)
        bi = jnp.min(jnp.where(x >= bm, ids[...], jnp.int32(n)), axis=0,
                     keepdims=True) + i * chunk
        better = bm > macc[...]
        iacc[...] = jnp.where(better, bi, iacc[...])
        macc[...] = jnp.maximum(bm, macc[...])

        @pl.when(i + _NBUF_IN < nchunks)
        def _():
            a_copy(i + _NBUF_IN, s).start()
            g_copy(i + _NBUF_IN, s).start()

        return carry

    jax.lax.fori_loop(0, nchunks, body_a, 0)

    def body_b(i, carry):
        s = jax.lax.rem(i, _NBUF_OUT)

        @pl.when(i >= _NBUF_OUT)
        def _():
            o_copy(i - _NBUF_OUT, s).wait()

        rel = iacc[...] - i * chunk
        o_buf[s] = (ids[...] == rel).astype(jnp.float32)
        o_copy(i, s).start()
        return carry

    jax.lax.fori_loop(0, nchunks, body_b, 0)
    for c in range(max(nchunks - _NBUF_OUT, 0), nchunks):
        o_copy(c, c % _NBUF_OUT).wait()


def kernel(alpha, gumbel, tau):
    del tau
    mm, kk = alpha.shape
    n, m = kk, mm
    chunk = _pick_chunk(n)
    inbuf = lambda: pltpu.VMEM((_NBUF_IN, chunk, m), jnp.float32)
    outbuf = pltpu.VMEM((_NBUF_OUT, chunk, m), jnp.float32)
    out_t = pl.pallas_call(
        functools.partial(_ring_kernel, n=n, m=m, chunk=chunk),
        in_specs=[
            pl.BlockSpec(memory_space=pl.ANY),
            pl.BlockSpec(memory_space=pl.ANY),
        ],
        out_specs=pl.BlockSpec(memory_space=pl.ANY),
        out_shape=jax.ShapeDtypeStruct((n, m), jnp.float32),
        scratch_shapes=[
            inbuf(), inbuf(), outbuf,
            pltpu.VMEM((chunk, m), jnp.int32),
            pltpu.VMEM((1, m), jnp.float32),
            pltpu.VMEM((1, m), jnp.int32),
            pltpu.SemaphoreType.DMA((_NBUF_IN,)),
            pltpu.SemaphoreType.DMA((_NBUF_IN,)),
            pltpu.SemaphoreType.DMA((_NBUF_OUT,)),
        ],
    )(alpha.T, gumbel.T)
    return out_t.T
